# Initial kernel scaffold; baseline (speedup 1.0000x reference)
import jax
import jax.numpy as jnp
from jax.experimental import pallas as pl


def kernel(padded, lens):
    raise NotImplementedError("write your pallas kernel here")


# speedup vs baseline: 1.3939x; 1.3939x over previous
import functools
import jax
import jax.numpy as jnp
from jax import lax
from jax.experimental import pallas as pl
from jax.experimental.pallas import tpu as pltpu
from jax.experimental.pallas import tpu_sc as plsc

T, B, D = 2048, 16, 512
T2H = T // 2
ROWS_PER_BLK = 2 * B
NW = 32
BLKS_PER_W = T2H // NW

PERM = [16 * (j % 2) + j // 2 for j in range(ROWS_PER_BLK)]

_mesh = plsc.VectorSubcoreMesh(
    core_axis_name="c", subcore_axis_name="s", num_cores=2, num_subcores=16
)


@functools.partial(
    pl.kernel,
    out_type=(
        jax.ShapeDtypeStruct((T2H * ROWS_PER_BLK, D), jnp.float32),
        jax.ShapeDtypeStruct((B,), jnp.int32),
    ),
    mesh=_mesh,
    scratch_types=[
        pltpu.VMEM((ROWS_PER_BLK + 1, D), jnp.float32),
        pltpu.VMEM((B,), jnp.int32),
        pltpu.VMEM((B,), jnp.int32),
        pltpu.SemaphoreType.DMA,
        pltpu.SemaphoreType.DMA,
    ],
)
def _shuffle(rows_hbm, lens_hbm, out_hbm, newlens_hbm, buf, lens_v, nl_v, in_sem, out_sem):
    wid = lax.axis_index("s") * 2 + lax.axis_index("c")

    pltpu.sync_copy(lens_hbm, lens_v)
    nl = lens_v[...] >> 1
    nl_v[...] = nl

    @pl.when(wid == 0)
    def _():
        pltpu.sync_copy(nl_v, newlens_hbm)

    zv = jnp.zeros((16,), jnp.float32)
    for i in range(D // 16):
        buf[ROWS_PER_BLK, pl.ds(i * 16, 16)] = zv

    def body(i, carry):
        t2 = wid * BLKS_PER_W + i
        base = t2 * ROWS_PER_BLK
        pltpu.async_copy(
            rows_hbm.at[pl.ds(base, ROWS_PER_BLK)],
            buf.at[pl.ds(0, ROWS_PER_BLK)],
            in_sem,
        ).wait()
        nlv = nl_v[...]
        waits = []
        for j in range(ROWS_PER_BLK):
            valid = t2 < nlv[j // 2]
            src = jnp.where(valid, PERM[j], ROWS_PER_BLK)
            waits.append(
                pltpu.async_copy(buf.at[src], out_hbm.at[base + j], out_sem)
            )
        for w in waits:
            w.wait()
        return carry

    lax.fori_loop(0, BLKS_PER_W, body, 0)


def kernel(padded, lens):
    rows = padded.reshape(T * B, D)
    out, newlens = _shuffle(rows, lens.astype(jnp.int32))
    return out.reshape(T2H, B, 2 * D), newlens
